# baseline (speedup 1.0000x reference)
import jax
import jax.numpy as jnp
from jax.experimental import pallas as pl
from jax.experimental.pallas import tpu as pltpu

TM = 512
NBUF = 4


def _top2_from_scores(scores, topi_ref, topv_ref):
    e = scores.shape[1]
    iota = jax.lax.broadcasted_iota(jnp.int32, scores.shape, 1)

    m1 = jnp.max(scores, axis=1, keepdims=True)
    i1 = jnp.min(jnp.where(scores == m1, iota, e), axis=1, keepdims=True)
    masked = jnp.where(iota == i1, -jnp.inf, scores)
    m2 = jnp.max(masked, axis=1, keepdims=True)
    i2 = jnp.min(jnp.where(masked == m2, iota, e), axis=1, keepdims=True)

    z = jnp.sum(jnp.exp(scores - m1), axis=1, keepdims=True)
    e2 = jnp.exp(m2 - m1)
    inv = 1.0 / (1.0 + e2 + 1e-9 * z)
    topi_ref[...] = jnp.concatenate([i1, i2], axis=1)
    topv_ref[...] = jnp.concatenate([inv, e2 * inv], axis=1)


def _router_block(x_hbm, wt_ref, topi_ref, topv_ref, xbuf, sem):
    i = pl.program_id(0)
    nblk = pl.num_programs(0)

    def start(j, slot):
        pltpu.make_async_copy(
            x_hbm.at[pl.ds(j * TM, TM), :], xbuf.at[slot], sem.at[slot]
        ).start()

    @pl.when(i == 0)
    def _():
        for j in range(NBUF - 1):
            start(j, j)

    @pl.when(i + NBUF - 1 < nblk)
    def _():
        j = i + NBUF - 1
        start(j, jax.lax.rem(j, NBUF))

    slot = jax.lax.rem(i, NBUF)
    pltpu.make_async_copy(
        x_hbm.at[pl.ds(i * TM, TM), :], xbuf.at[slot], sem.at[slot]
    ).wait()

    scores = jax.lax.dot_general(
        xbuf[slot], wt_ref[...], (((1,), (0,)), ((), ())),
        preferred_element_type=jnp.float32)
    _top2_from_scores(scores, topi_ref, topv_ref)


@jax.jit
def kernel(x, W):
    tokens, d = x.shape
    n_exp = W.shape[0]
    wt = W.T
    grid = (tokens // TM,)
    topi, topv = pl.pallas_call(
        _router_block,
        grid=grid,
        in_specs=[
            pl.BlockSpec(memory_space=pltpu.MemorySpace.HBM),
            pl.BlockSpec((d, n_exp), lambda i: (0, 0)),
        ],
        out_specs=[
            pl.BlockSpec((TM, 2), lambda i: (i, 0)),
            pl.BlockSpec((TM, 2), lambda i: (i, 0)),
        ],
        out_shape=[
            jax.ShapeDtypeStruct((tokens, 2), jnp.int32),
            jax.ShapeDtypeStruct((tokens, 2), jnp.float32),
        ],
        scratch_shapes=[
            pltpu.VMEM((NBUF, TM, d), jnp.float32),
            pltpu.SemaphoreType.DMA((NBUF,)),
        ],
    )(x, wt)
    return (topi, topv)


# speedup vs baseline: 1.0094x; 1.0094x over previous
import jax
import jax.numpy as jnp
from jax.experimental import pallas as pl

TM = 1024


def _router_block(x_ref, wt_ref, topi_ref, topv_ref):
    scores = jax.lax.dot_general(
        x_ref[...], wt_ref[...], (((1,), (0,)), ((), ())),
        preferred_element_type=jnp.float32)
    e = scores.shape[1]
    iota = jax.lax.broadcasted_iota(jnp.int32, scores.shape, 1)

    m1 = jnp.max(scores, axis=1, keepdims=True)
    i1 = jnp.min(jnp.where(scores == m1, iota, e), axis=1, keepdims=True)
    masked = jnp.where(iota == i1, -jnp.inf, scores)
    m2 = jnp.max(masked, axis=1, keepdims=True)
    i2 = jnp.min(jnp.where(masked == m2, iota, e), axis=1, keepdims=True)

    z = jnp.sum(jnp.exp(scores - m1), axis=1, keepdims=True)
    e2 = jnp.exp(m2 - m1)
    inv = 1.0 / (1.0 + e2 + 1e-9 * z)
    topi_ref[...] = jnp.concatenate([i1, i2], axis=1)
    topv_ref[...] = jnp.concatenate([inv, e2 * inv], axis=1)


@jax.jit
def kernel(x, W):
    tokens, d = x.shape
    n_exp = W.shape[0]
    wt = W.T
    grid = (tokens // TM,)
    topi, topv = pl.pallas_call(
        _router_block,
        grid=grid,
        in_specs=[
            pl.BlockSpec((TM, d), lambda i: (i, 0)),
            pl.BlockSpec((d, n_exp), lambda i: (0, 0)),
        ],
        out_specs=[
            pl.BlockSpec((TM, 2), lambda i: (i, 0)),
            pl.BlockSpec((TM, 2), lambda i: (i, 0)),
        ],
        out_shape=[
            jax.ShapeDtypeStruct((tokens, 2), jnp.int32),
            jax.ShapeDtypeStruct((tokens, 2), jnp.float32),
        ],
    )(x, wt)
    return (topi, topv)


# speedup vs baseline: 1.0109x; 1.0015x over previous
import jax
import jax.numpy as jnp
from jax.experimental import pallas as pl

TM = 512


def _top2(scores, topi_ref, topv_ref):
    e = scores.shape[1]
    iota = jax.lax.broadcasted_iota(jnp.int32, scores.shape, 1)
    m1 = jnp.max(scores, axis=1, keepdims=True)
    i1 = jnp.min(jnp.where(scores == m1, iota, e), axis=1, keepdims=True)
    masked = jnp.where(iota == i1, -jnp.inf, scores)
    m2 = jnp.max(masked, axis=1, keepdims=True)
    i2 = jnp.min(jnp.where(masked == m2, iota, e), axis=1, keepdims=True)
    z = jnp.sum(jnp.exp(scores - m1), axis=1, keepdims=True)
    e2 = jnp.exp(m2 - m1)
    inv = 1.0 / (1.0 + e2 + 1e-9 * z)
    topi_ref[...] = jnp.concatenate([i1, i2], axis=1)
    topv_ref[...] = jnp.concatenate([inv, e2 * inv], axis=1)


def _router_block(xa_ref, xb_ref, wt_ref, ia_ref, ib_ref, va_ref, vb_ref):
    dn = (((1,), (0,)), ((), ()))
    sa = jax.lax.dot_general(xa_ref[...], wt_ref[...], dn,
                             preferred_element_type=jnp.float32)
    _top2(sa, ia_ref, va_ref)
    sb = jax.lax.dot_general(xb_ref[...], wt_ref[...], dn,
                             preferred_element_type=jnp.float32)
    _top2(sb, ib_ref, vb_ref)


@jax.jit
def kernel(x, W):
    tokens, d = x.shape
    n_exp = W.shape[0]
    wt = W.T
    half = tokens // 2
    nblk = half // TM
    ia, ib, va, vb = pl.pallas_call(
        _router_block,
        grid=(nblk,),
        in_specs=[
            pl.BlockSpec((TM, d), lambda i: (i, 0)),
            pl.BlockSpec((TM, d), lambda i, nblk=nblk: (i + nblk, 0)),
            pl.BlockSpec((d, n_exp), lambda i: (0, 0)),
        ],
        out_specs=[
            pl.BlockSpec((TM, 2), lambda i: (i, 0)),
            pl.BlockSpec((TM, 2), lambda i: (i, 0)),
            pl.BlockSpec((TM, 2), lambda i: (i, 0)),
            pl.BlockSpec((TM, 2), lambda i: (i, 0)),
        ],
        out_shape=[
            jax.ShapeDtypeStruct((half, 2), jnp.int32),
            jax.ShapeDtypeStruct((half, 2), jnp.int32),
            jax.ShapeDtypeStruct((half, 2), jnp.float32),
            jax.ShapeDtypeStruct((half, 2), jnp.float32),
        ],
    )(x, x, wt)
    return (jnp.concatenate([ia, ib], 0), jnp.concatenate([va, vb], 0))
